# baseline (speedup 1.0000x reference)
import functools

import jax
import jax.numpy as jnp
from jax import lax
from jax.experimental import pallas as pl
from jax.experimental.pallas import tpu as pltpu
from jax.experimental.pallas import tpu_sc as plsc

_ROWS = 128
_COLS = 32768
_NC = 2
_NS = 16
_NW = _NC * _NS
_RPW = _ROWS // _NW
_CHUNK = 8192
_NCHUNK = _COLS // _CHUNK

_mesh = plsc.VectorSubcoreMesh(core_axis_name="c", subcore_axis_name="s")


@functools.partial(
    pl.kernel,
    mesh=_mesh,
    out_type=jax.ShapeDtypeStruct((_ROWS, _COLS), jnp.float32),
    scratch_types=[
        pltpu.VMEM((2, _CHUNK), jnp.float32),
        pltpu.VMEM((2, _COLS), jnp.float32),
        pltpu.SemaphoreType.DMA,
        pltpu.SemaphoreType.DMA,
        pltpu.SemaphoreType.DMA,
        pltpu.SemaphoreType.DMA,
    ],
    compiler_params=pltpu.CompilerParams(needs_layout_passes=False),
)
def _sc_rank_onehot(scores_hbm, out_hbm, inbuf, orow, isem0, isem1,
                    osem0, osem1):
    isems = (isem0, isem1)
    osems = (osem0, osem1)
    cid = lax.axis_index("c")
    sid = lax.axis_index("s")
    wid = sid * _NC + cid
    base = wid * _RPW

    def in_copy(g, buf):
        row = base + g // _NCHUNK
        off = (g % _NCHUNK) * _CHUNK
        return pltpu.make_async_copy(
            scores_hbm.at[row, pl.ds(off, _CHUNK)],
            inbuf.at[buf],
            isems[buf],
        )

    in_copy(0, 0).start()

    zeros16 = jnp.zeros((16,), jnp.float32)

    def zero_body(i, _):
        orow[0, pl.ds(i * 16, 16)] = zeros16
        orow[1, pl.ds(i * 16, 16)] = zeros16
        return 0

    lax.fori_loop(0, _COLS // 16, zero_body, 0, unroll=4)

    def count_chunk(buf, pivot, acc):
        def body(i, a):
            v = inbuf[buf, pl.ds(i * 16, 16)]
            return a + plsc.all_reduce_population_count(v > pivot)
        return lax.fori_loop(0, _CHUNK // 16, body, acc, unroll=8)

    j0_saved = [None] * _RPW
    for r in range(_RPW):
        row = base + r
        acc = jnp.zeros((16,), jnp.int32)
        pivot = jnp.float32(0)
        for ch in range(_NCHUNK):
            g = r * _NCHUNK + ch
            if g + 1 < _RPW * _NCHUNK:
                in_copy(g + 1, (g + 1) % 2).start()
            in_copy(g, g % 2).wait()
            if ch == 0:
                pivot = inbuf[g % 2, pl.ds(0, 16)][0]
            acc = count_chunk(g % 2, pivot, acc)
        count = acc[0]
        j0 = count & ~15
        lane = count & 15
        hot = jnp.where(lax.iota(jnp.int32, 16) == lane,
                        jnp.float32(1), jnp.float32(0))
        ob = r % 2
        out_cp = pltpu.make_async_copy(orow.at[ob], out_hbm.at[row], osems[ob])
        if r >= 2:
            pltpu.make_async_copy(
                orow.at[ob], out_hbm.at[base + r - 2], osems[ob]).wait()
            orow[ob, pl.ds(j0_saved[r - 2], 16)] = zeros16
        orow[ob, pl.ds(j0, 16)] = hot
        out_cp.start()
        j0_saved[r] = j0

    pltpu.make_async_copy(
        orow.at[(_RPW - 2) % 2], out_hbm.at[base + _RPW - 2],
        osems[(_RPW - 2) % 2]).wait()
    pltpu.make_async_copy(
        orow.at[(_RPW - 1) % 2], out_hbm.at[base + _RPW - 1],
        osems[(_RPW - 1) % 2]).wait()


def kernel(scores):
    return _sc_rank_onehot(scores)


# speedup vs baseline: 1.0522x; 1.0522x over previous
import functools

import jax
import jax.numpy as jnp
from jax import lax
from jax.experimental import pallas as pl
from jax.experimental.pallas import tpu as pltpu
from jax.experimental.pallas import tpu_sc as plsc

_ROWS = 128
_COLS = 32768
_NC = 2
_NS = 16
_NW = _NC * _NS
_RPW = _ROWS // _NW
_CHUNK = 16384
_NCHUNK = _COLS // _CHUNK
_NGLOBAL = _RPW * _NCHUNK

_mesh = plsc.VectorSubcoreMesh(core_axis_name="c", subcore_axis_name="s")


@functools.partial(
    pl.kernel,
    mesh=_mesh,
    out_type=jax.ShapeDtypeStruct((_ROWS, _COLS), jnp.float32),
    scratch_types=[
        pltpu.VMEM((2, _CHUNK), jnp.float32),
        pltpu.VMEM((2, _COLS), jnp.float32),
        pltpu.SemaphoreType.DMA,
        pltpu.SemaphoreType.DMA,
        pltpu.SemaphoreType.DMA,
        pltpu.SemaphoreType.DMA,
    ],
    compiler_params=pltpu.CompilerParams(needs_layout_passes=False),
)
def _sc_rank_onehot(scores_hbm, out_hbm, inbuf, orow, isem0, isem1,
                    osem0, osem1):
    isems = (isem0, isem1)
    osems = (osem0, osem1)
    cid = lax.axis_index("c")
    sid = lax.axis_index("s")
    wid = sid * _NC + cid
    base = wid * _RPW

    def in_copy(g, buf):
        row = base + g // _NCHUNK
        off = (g % _NCHUNK) * _CHUNK
        return pltpu.make_async_copy(
            scores_hbm.at[row, pl.ds(off, _CHUNK)],
            inbuf.at[buf],
            isems[buf],
        )

    in_copy(0, 0).start()

    zeros16 = jnp.zeros((16,), jnp.float32)

    def zero_body(i, _):
        orow[0, pl.ds(i * 16, 16)] = zeros16
        orow[1, pl.ds(i * 16, 16)] = zeros16
        return 0

    lax.fori_loop(0, _COLS // 16, zero_body, 0, unroll=8)

    def count_chunk(buf, pivot, acc):
        def body(i, a):
            v = inbuf[buf, pl.ds(i * 16, 16)]
            return a + plsc.all_reduce_population_count(v > pivot)
        return lax.fori_loop(0, _CHUNK // 16, body, acc, unroll=16)

    j0_saved = [None] * _RPW
    for r in range(_RPW):
        row = base + r
        acc = jnp.zeros((16,), jnp.int32)
        pivot = jnp.float32(0)
        for ch in range(_NCHUNK):
            g = r * _NCHUNK + ch
            if g + 1 < _NGLOBAL:
                in_copy(g + 1, (g + 1) % 2).start()
            in_copy(g, g % 2).wait()
            if ch == 0:
                pivot = inbuf[g % 2, pl.ds(0, 16)][0]
            acc = count_chunk(g % 2, pivot, acc)
        count = acc[0]
        j0 = count & ~15
        lane = count & 15
        hot = jnp.where(lax.iota(jnp.int32, 16) == lane,
                        jnp.float32(1), jnp.float32(0))
        ob = r % 2
        out_cp = pltpu.make_async_copy(orow.at[ob], out_hbm.at[row], osems[ob])
        if r >= 2:
            pltpu.make_async_copy(
                orow.at[ob], out_hbm.at[base + r - 2], osems[ob]).wait()
            orow[ob, pl.ds(j0_saved[r - 2], 16)] = zeros16
        orow[ob, pl.ds(j0, 16)] = hot
        out_cp.start()
        j0_saved[r] = j0

    pltpu.make_async_copy(
        orow.at[(_RPW - 2) % 2], out_hbm.at[base + _RPW - 2],
        osems[(_RPW - 2) % 2]).wait()
    pltpu.make_async_copy(
        orow.at[(_RPW - 1) % 2], out_hbm.at[base + _RPW - 1],
        osems[(_RPW - 1) % 2]).wait()


def kernel(scores):
    return _sc_rank_onehot(scores)
